# baseline (speedup 1.0000x reference)
import functools

import jax
import jax.numpy as jnp
from jax import lax
from jax.experimental import pallas as pl
from jax.experimental.pallas import tpu as pltpu
from jax.experimental.pallas import tpu_sc as plsc

BATCH = 4096
HIST = 200
EMBED_DIM = 64
B = BATCH * HIST
NC = 2
NS = 16
NW = NC * NS
BPW = B // NW
CHUNK = 512
NCHUNK = BPW // CHUNK


def _build_kernel():
  mesh = plsc.VectorSubcoreMesh(core_axis_name="c", subcore_axis_name="s")

  @functools.partial(
      pl.kernel,
      mesh=mesh,
      out_type=jax.ShapeDtypeStruct((B, EMBED_DIM), jnp.float32),
      scratch_types=[
          pltpu.VMEM((CHUNK,), jnp.int32),
          pltpu.VMEM((CHUNK, EMBED_DIM), jnp.float32),
          pltpu.SemaphoreType.DMA,
      ],
      compiler_params=pltpu.CompilerParams(use_tc_tiling_on_sc=False),
  )
  def gather_kernel(idx_hbm, table_hbm, out_hbm, idx_v, rows_v, sem):
    wid = lax.axis_index("s") * NC + lax.axis_index("c")
    base = wid * BPW

    def body(i, carry):
      off = base + i * CHUNK
      pltpu.sync_copy(idx_hbm.at[pl.ds(off, CHUNK)], idx_v)
      pltpu.async_copy(table_hbm.at[idx_v], rows_v, sem).wait()
      pltpu.sync_copy(rows_v, out_hbm.at[pl.ds(off, CHUNK)])
      return carry

    lax.fori_loop(0, NCHUNK, body, 0)

  return gather_kernel


_GATHER = _build_kernel()


def kernel(input_x, table):
  idx = input_x.reshape(B).astype(jnp.int32)
  out = _GATHER(idx, table)
  return out.reshape(BATCH, HIST, EMBED_DIM)


# speedup vs baseline: 1.0445x; 1.0445x over previous
import functools

import jax
import jax.numpy as jnp
from jax import lax
from jax.experimental import pallas as pl
from jax.experimental.pallas import tpu as pltpu
from jax.experimental.pallas import tpu_sc as plsc

BATCH = 4096
HIST = 200
EMBED_DIM = 64
B = BATCH * HIST
NC = 2
NS = 16
NW = NC * NS
BPW = B // NW
CHUNK = 512
NCHUNK = BPW // CHUNK
NPAIR = NCHUNK // 2


def _build_kernel():
  mesh = plsc.VectorSubcoreMesh(core_axis_name="c", subcore_axis_name="s")

  @functools.partial(
      pl.kernel,
      mesh=mesh,
      out_type=jax.ShapeDtypeStruct((NW, NCHUNK, CHUNK, EMBED_DIM),
                                    jnp.float32),
      scratch_types=[
          pltpu.VMEM((BPW,), jnp.int32),
          pltpu.VMEM((CHUNK, EMBED_DIM), jnp.float32),
          pltpu.VMEM((CHUNK, EMBED_DIM), jnp.float32),
          pltpu.SemaphoreType.DMA,
          pltpu.SemaphoreType.DMA,
      ],
      compiler_params=pltpu.CompilerParams(use_tc_tiling_on_sc=False),
  )
  def gather_kernel(idx_hbm, table_hbm, out_hbm, idx_v, rows0, rows1,
                    sem0, sem1):
    wid = lax.axis_index("s") * NC + lax.axis_index("c")
    pltpu.sync_copy(idx_hbm.at[wid], idx_v)

    def gather_start(j, buf, sem):
      pltpu.async_copy(
          table_hbm.at[idx_v.at[pl.ds(j * CHUNK, CHUNK)]], buf, sem)

    def gather_wait(j, buf, sem):
      pltpu.make_async_copy(
          table_hbm.at[idx_v.at[pl.ds(j * CHUNK, CHUNK)]], buf, sem).wait()

    gather_start(0, rows0, sem0)
    gather_start(1, rows1, sem1)

    def body(i, carry):
      for b, (buf, sem) in enumerate(((rows0, sem0), (rows1, sem1))):
        j = 2 * i + b
        gather_wait(j, buf, sem)
        pltpu.sync_copy(buf, out_hbm.at[wid, j])

        @pl.when(i < NPAIR - 1)
        def _():
          gather_start(j + 2, buf, sem)
      return carry

    lax.fori_loop(0, NPAIR, body, 0)

  return gather_kernel


_GATHER = _build_kernel()


def kernel(input_x, table):
  idx = input_x.reshape(NW, BPW).astype(jnp.int32)
  out = _GATHER(idx, table)
  return out.reshape(BATCH, HIST, EMBED_DIM)


# speedup vs baseline: 1.2746x; 1.2203x over previous
import functools

import jax
import jax.numpy as jnp
from jax import lax
from jax.experimental import pallas as pl
from jax.experimental.pallas import tpu as pltpu
from jax.experimental.pallas import tpu_sc as plsc

BATCH = 4096
HIST = 200
EMBED_DIM = 64
PADDED_DIM = 128
B = BATCH * HIST
NC = 2
NS = 16
NW = NC * NS
BPW = B // NW
CHUNK = 256
NCHUNK = BPW // CHUNK
NPAIR = NCHUNK // 2


def _build_kernel():
  mesh = plsc.VectorSubcoreMesh(core_axis_name="c", subcore_axis_name="s")

  @functools.partial(
      pl.kernel,
      mesh=mesh,
      out_type=jax.ShapeDtypeStruct((B, PADDED_DIM), jnp.float32),
      scratch_types=[
          pltpu.VMEM((BPW,), jnp.int32),
          pltpu.VMEM((CHUNK, PADDED_DIM), jnp.float32),
          pltpu.VMEM((CHUNK, PADDED_DIM), jnp.float32),
          pltpu.SemaphoreType.DMA,
          pltpu.SemaphoreType.DMA,
      ],
      compiler_params=pltpu.CompilerParams(use_tc_tiling_on_sc=True),
  )
  def gather_kernel(idx_hbm, table_hbm, out_hbm, idx_v, rows0, rows1,
                    sem0, sem1):
    wid = lax.axis_index("s") * NC + lax.axis_index("c")
    base = wid * BPW
    pltpu.sync_copy(idx_hbm.at[pl.ds(base, BPW)], idx_v)

    def gather_start(j, buf, sem):
      pltpu.async_copy(
          table_hbm.at[idx_v.at[pl.ds(j * CHUNK, CHUNK)]], buf, sem)

    def gather_wait(j, buf, sem):
      pltpu.make_async_copy(
          table_hbm.at[idx_v.at[pl.ds(j * CHUNK, CHUNK)]], buf, sem).wait()

    gather_start(0, rows0, sem0)
    gather_start(1, rows1, sem1)

    def body(i, carry):
      for b, (buf, sem) in enumerate(((rows0, sem0), (rows1, sem1))):
        j = 2 * i + b
        gather_wait(j, buf, sem)
        pltpu.sync_copy(buf, out_hbm.at[pl.ds(base + j * CHUNK, CHUNK)])

        @pl.when(i < NPAIR - 1)
        def _():
          gather_start(j + 2, buf, sem)
      return carry

    lax.fori_loop(0, NPAIR, body, 0)

  return gather_kernel


_GATHER = _build_kernel()


def kernel(input_x, table):
  idx = input_x.reshape(B).astype(jnp.int32)
  table_padded = jnp.pad(table, ((0, 0), (0, PADDED_DIM - EMBED_DIM)))
  out = _GATHER(idx, table_padded)
  return out[:, :EMBED_DIM].reshape(BATCH, HIST, EMBED_DIM)
